# Initial kernel scaffold; baseline (speedup 1.0000x reference)
import functools

import jax
import jax.numpy as jnp
from jax import lax
from jax.experimental import pallas as pl
from jax.experimental.pallas import tpu as pltpu
from jax.experimental.pallas import tpu_sc as plsc

VOCAB = 100000
DIM = 64
BATCH = 4096
SEQ = 200

_INFO = plsc.get_sparse_core_info()
_NC, _NS = _INFO.num_cores, _INFO.num_subcores
_NW = _NC * _NS
_TOTAL = BATCH * SEQ
_CHUNK = 128
_N_CHUNKS = _TOTAL // _CHUNK
_CHUNKS_PER_W = _N_CHUNKS // _NW


def _body(idx_hbm, table_hbm, out_hbm, idx_v, rows_v, sem):
    wid = lax.axis_index("s") * _NC + lax.axis_index("c")
    row0 = wid * _CHUNKS_PER_W
    pltpu.sync_copy(idx_hbm.at[pl.ds(row0, _CHUNKS_PER_W)], idx_v)

    def chunk(j, carry):
        pltpu.async_copy(table_hbm.at[idx_v.at[j]], rows_v, sem).wait()
        pltpu.sync_copy(rows_v, out_hbm.at[pl.ds((row0 + j) * _CHUNK, _CHUNK)])
        return carry

    lax.fori_loop(0, _CHUNKS_PER_W, chunk, 0)


def kernel(inputs, table):
    idx = inputs.reshape(_N_CHUNKS, _CHUNK)
    run = pl.kernel(
        _body,
        mesh=plsc.VectorSubcoreMesh(core_axis_name="c", subcore_axis_name="s"),
        out_type=jax.ShapeDtypeStruct((_TOTAL, DIM), jnp.float32),
        scratch_types=[
            pltpu.VMEM((_CHUNKS_PER_W, _CHUNK), jnp.int32),
            pltpu.VMEM((_CHUNK, DIM), jnp.float32),
            pltpu.SemaphoreType.DMA,
        ],
    )
    out = run(idx, table)
    return out.reshape(BATCH, SEQ, DIM)


# speedup vs baseline: 3.5484x; 3.5484x over previous
import functools

import jax
import jax.numpy as jnp
from jax import lax
from jax.experimental import pallas as pl
from jax.experimental.pallas import tpu as pltpu
from jax.experimental.pallas import tpu_sc as plsc

VOCAB = 100000
DIM = 64
BATCH = 4096
SEQ = 200

_INFO = plsc.get_sparse_core_info()
_NC, _NS = _INFO.num_cores, _INFO.num_subcores
_NW = _NC * _NS
_TOTAL = BATCH * SEQ
_CHUNK = 128
_N_CHUNKS = _TOTAL // _CHUNK
_CHUNKS_PER_W = _N_CHUNKS // _NW


def _body(idx_hbm, table_hbm, out_hbm, idx_v, rows_v, sem):
    wid = lax.axis_index("s") * _NC + lax.axis_index("c")
    row0 = wid * _CHUNKS_PER_W
    pltpu.sync_copy(idx_hbm.at[pl.ds(row0, _CHUNKS_PER_W)], idx_v)

    def chunk(j, carry):
        pltpu.async_copy(table_hbm.at[idx_v.at[j]], rows_v, sem).wait()
        pltpu.sync_copy(rows_v, out_hbm.at[pl.ds((row0 + j) * _CHUNK, _CHUNK)])
        return carry

    lax.fori_loop(0, _CHUNKS_PER_W, chunk, 0)


def kernel(inputs, table):
    idx = inputs.reshape(_N_CHUNKS, _CHUNK)
    run = pl.kernel(
        _body,
        mesh=plsc.VectorSubcoreMesh(core_axis_name="c", subcore_axis_name="s"),
        out_type=jax.ShapeDtypeStruct((_TOTAL, DIM), jnp.float32),
        scratch_types=[
            pltpu.VMEM((_CHUNKS_PER_W, _CHUNK), jnp.int32),
            pltpu.VMEM((_CHUNK, DIM), jnp.float32),
            pltpu.SemaphoreType.DMA,
        ],
        compiler_params=pltpu.CompilerParams(use_tc_tiling_on_sc=False),
    )
    out = run(idx, table)
    return out.reshape(BATCH, SEQ, DIM)


# speedup vs baseline: 4.2544x; 1.1990x over previous
import functools

import jax
import jax.numpy as jnp
from jax import lax
from jax.experimental import pallas as pl
from jax.experimental.pallas import tpu as pltpu
from jax.experimental.pallas import tpu_sc as plsc

VOCAB = 100000
DIM = 64
BATCH = 4096
SEQ = 200

_INFO = plsc.get_sparse_core_info()
_NC, _NS = _INFO.num_cores, _INFO.num_subcores
_NW = _NC * _NS
_TOTAL = BATCH * SEQ
_CHUNK = 128
_N_CHUNKS = _TOTAL // _CHUNK
_CHUNKS_PER_W = _N_CHUNKS // _NW


_NBUF = 8
_GDEPTH = 4


def _body(idx_hbm, table_hbm, out_hbm, idx_v, rows_v, *sems):
    gsem = sems[:_NBUF]
    ssem = sems[_NBUF:]
    wid = lax.axis_index("s") * _NC + lax.axis_index("c")
    row0 = wid * _CHUNKS_PER_W
    pltpu.sync_copy(idx_hbm.at[pl.ds(row0, _CHUNKS_PER_W)], idx_v)

    def gather(g, b):
        pltpu.async_copy(table_hbm.at[idx_v.at[g]], rows_v.at[b], gsem[b])

    def store(g, b):
        pltpu.async_copy(
            rows_v.at[b], out_hbm.at[pl.ds((row0 + g) * _CHUNK, _CHUNK)], ssem[b]
        )

    def wait_store(b):
        pltpu.make_async_copy(
            rows_v.at[b], out_hbm.at[pl.ds(b * _CHUNK, _CHUNK)], ssem[b]
        ).wait()

    def wait_gather(g, b):
        pltpu.make_async_copy(
            table_hbm.at[idx_v.at[g]], rows_v.at[b], gsem[b]
        ).wait()

    for g in range(_GDEPTH):
        gather(g, g % _NBUF)

    def step(k, carry):
        for b in range(_NBUF):
            g = k * _NBUF + b
            gn = g + _GDEPTH
            bn = (b + _GDEPTH) % _NBUF

            @pl.when(gn >= _NBUF)
            def _():
                wait_store(bn)

            @pl.when(gn < _CHUNKS_PER_W)
            def _():
                gather(gn, bn)

            wait_gather(g, b)
            store(g, b)
        return carry

    lax.fori_loop(0, _CHUNKS_PER_W // _NBUF, step, 0)

    for i in range(_NBUF - _GDEPTH):
        wait_store((_CHUNKS_PER_W - (_NBUF - _GDEPTH) + i) % _NBUF)


def kernel(inputs, table):
    idx = inputs.reshape(_N_CHUNKS, _CHUNK)
    run = pl.kernel(
        _body,
        mesh=plsc.VectorSubcoreMesh(core_axis_name="c", subcore_axis_name="s"),
        out_type=jax.ShapeDtypeStruct((_TOTAL, DIM), jnp.float32),
        scratch_types=[
            pltpu.VMEM((_CHUNKS_PER_W, _CHUNK), jnp.int32),
            pltpu.VMEM((_NBUF, _CHUNK, DIM), jnp.float32),
        ]
        + [pltpu.SemaphoreType.DMA] * (2 * _NBUF),
        compiler_params=pltpu.CompilerParams(use_tc_tiling_on_sc=False),
    )
    out = run(idx, table)
    return out.reshape(BATCH, SEQ, DIM)


# speedup vs baseline: 5.9289x; 1.3936x over previous
import jax
import jax.numpy as jnp
from jax import lax
from jax.experimental import pallas as pl
from jax.experimental.pallas import tpu as pltpu
from jax.experimental.pallas import tpu_sc as plsc

VOCAB = 100000
DIM = 64
BATCH = 4096
SEQ = 200

_INFO = plsc.get_sparse_core_info()
_NC, _NS = _INFO.num_cores, _INFO.num_subcores
_NW = _NC * _NS
_BT = BATCH // _NW
_L = 16

_NR = 4
_NT = 2
_GD = 2
_OUTER = SEQ // _NR


def _body(idx_hbm, table_hbm, out_hbm, idx_v, rows_v, tp_v, *sems):
    gsem = sems[:_NR]
    ssem = sems[_NR:]
    wid = lax.axis_index("s") * _NC + lax.axis_index("c")
    bt = wid
    pltpu.sync_copy(idx_hbm.at[pl.ds(0, SEQ), pl.ds(bt * _BT, _BT)], idx_v)

    def gather(s, rb):
        pltpu.async_copy(table_hbm.at[idx_v.at[s]], rows_v.at[rb], gsem[rb])

    def wait_gather(rb):
        pltpu.make_async_copy(
            table_hbm.at[idx_v.at[0]], rows_v.at[rb], gsem[rb]
        ).wait()

    def store(s, tb):
        pltpu.async_copy(
            tp_v.at[tb], out_hbm.at[s, pl.ds(0, DIM // 8), bt], ssem[tb]
        )

    def wait_store(tb):
        pltpu.make_async_copy(
            tp_v.at[tb], out_hbm.at[0, pl.ds(0, DIM // 8), bt], ssem[tb]
        ).wait()

    iota = lax.broadcasted_iota(jnp.int32, (_L,), 0)
    diag = [(iota + k) & (_L - 1) for k in range(_L)]

    def transpose(rb, tb):
        def tile(j, carry):
            for m in range(4):
                t = j * 4 + m
                d0 = (t // 8) * _L
                c0 = (t % 8) * _L
                ri = iota + c0
                for k in range(_L):
                    ci = diag[k] + d0
                    v = plsc.load_gather(rows_v.at[rb], [ri, ci])
                    plsc.store_scatter(
                        tp_v.at[tb], [ci >> 3, ci & 7, ri], v
                    )
            return carry

        lax.fori_loop(0, 8, tile, 0)

    for s in range(_GD):
        gather(s, s % _NR)

    def step(k, carry):
        for u in range(_NR):
            rb = u
            tb = u % _NT
            rn = (u + _GD) % _NR
            s = k * _NR + u

            if u + _GD < _NR:
                gather(s + _GD, rn)
            else:
                @pl.when(k < _OUTER - 1)
                def _():
                    gather(s + _GD, rn)

            wait_gather(rb)

            if u < _NT:
                @pl.when(k > 0)
                def _():
                    wait_store(tb)
            else:
                wait_store(tb)

            transpose(rb, tb)
            store(s, tb)
        return carry

    lax.fori_loop(0, _OUTER, step, 0)

    for tb in range(_NT):
        wait_store((_OUTER * _NR - _NT + tb) % _NT)


def kernel(inputs, table):
    run = pl.kernel(
        _body,
        mesh=plsc.VectorSubcoreMesh(core_axis_name="c", subcore_axis_name="s"),
        out_type=jax.ShapeDtypeStruct(
            (SEQ, DIM // 8, _NW, 8, _BT), jnp.float32
        ),
        scratch_types=[
            pltpu.VMEM((SEQ, _BT), jnp.int32),
            pltpu.VMEM((_NR, _BT, DIM), jnp.float32),
            pltpu.VMEM((_NT, DIM // 8, 8, _BT), jnp.float32),
        ]
        + [pltpu.SemaphoreType.DMA] * (_NR + _NT),
        compiler_params=pltpu.CompilerParams(
            use_tc_tiling_on_sc=False, needs_layout_passes=False
        ),
    )
    out5 = run(jnp.transpose(inputs), table)
    return out5.transpose(2, 4, 0, 1, 3).reshape(BATCH, SEQ, DIM)


# speedup vs baseline: 11.5581x; 1.9495x over previous
import jax
import jax.numpy as jnp
from jax import lax
from jax.experimental import pallas as pl
from jax.experimental.pallas import tpu as pltpu
from jax.experimental.pallas import tpu_sc as plsc

VOCAB = 100000
DIM = 64
BATCH = 4096
SEQ = 200

_INFO = plsc.get_sparse_core_info()
_NC, _NS = _INFO.num_cores, _INFO.num_subcores
_NW = _NC * _NS
_BT = BATCH // _NW
_L = 16

_NB = 5
_GD = 3
_OUTER = SEQ // _NB


def _body(idx_hbm, table_hbm, out_hbm, idx_v, rows_v, tp_v, *sems):
    gsem = sems[:_NB]
    ssem = sems[_NB:]
    wid = lax.axis_index("s") * _NC + lax.axis_index("c")
    bt = wid
    pltpu.sync_copy(idx_hbm.at[pl.ds(0, SEQ), pl.ds(bt * _BT, _BT)], idx_v)

    def gather(s, rb):
        pltpu.async_copy(table_hbm.at[idx_v.at[s]], rows_v.at[rb], gsem[rb])

    def wait_gather(rb):
        pltpu.make_async_copy(
            table_hbm.at[idx_v.at[0]], rows_v.at[rb], gsem[rb]
        ).wait()

    def store(s, tb):
        for dt in range(DIM // 8):
            pltpu.async_copy(
                tp_v.at[tb, pl.ds(dt * 8, 8)],
                out_hbm.at[s, dt, bt],
                ssem[tb],
            )

    def wait_store(tb):
        for dt in range(DIM // 8):
            pltpu.make_async_copy(
                tp_v.at[tb, pl.ds(dt * 8, 8)],
                out_hbm.at[0, dt, bt],
                ssem[tb],
            ).wait()

    iota = lax.broadcasted_iota(jnp.int32, (_L,), 0)
    diag = [(iota + k) & (_L - 1) for k in range(_L)]

    def transpose(rb, tb):
        @plsc.parallel_loop(0, 32, unroll=4)
        def tile(t):
            d0 = (t // 8) * _L
            c0 = (t % 8) * _L
            ri = iota + c0
            for k in range(_L):
                ci = diag[k] + d0
                v = plsc.load_gather(rows_v.at[rb], [ri, ci])
                plsc.store_scatter(tp_v.at[tb], [ci, ri], v)

    for s in range(_GD):
        gather(s, s % _NB)

    def step(k, carry):
        for u in range(_NB):
            b = u
            bn = (u + _GD) % _NB
            s = k * _NB + u

            if u + _GD < _NB:
                gather(s + _GD, bn)
            else:
                @pl.when(k < _OUTER - 1)
                def _():
                    gather(s + _GD, bn)

            wait_gather(b)

            @pl.when(k > 0)
            def _():
                wait_store(b)

            transpose(b, b)
            store(s, b)
        return carry

    lax.fori_loop(0, _OUTER, step, 0)

    for tb in range(_NB):
        wait_store(tb)


def kernel(inputs, table):
    run = pl.kernel(
        _body,
        mesh=plsc.VectorSubcoreMesh(core_axis_name="c", subcore_axis_name="s"),
        out_type=jax.ShapeDtypeStruct(
            (SEQ, DIM // 8, _NW, 8, _BT), jnp.float32
        ),
        scratch_types=[
            pltpu.VMEM((SEQ, _BT), jnp.int32),
            pltpu.VMEM((_NB, _BT, DIM), jnp.float32),
            pltpu.VMEM((_NB, DIM, _BT), jnp.float32),
        ]
        + [pltpu.SemaphoreType.DMA] * (2 * _NB),
        compiler_params=pltpu.CompilerParams(
            use_tc_tiling_on_sc=False,
            needs_layout_passes=False,
            disable_bounds_checks=True,
        ),
    )
    out5 = run(jnp.transpose(inputs), table)
    return out5.transpose(2, 4, 0, 1, 3).reshape(BATCH, SEQ, DIM)


# speedup vs baseline: 13.4687x; 1.1653x over previous
import jax
import jax.numpy as jnp
from jax import lax
from jax.experimental import pallas as pl
from jax.experimental.pallas import tpu as pltpu
from jax.experimental.pallas import tpu_sc as plsc

VOCAB = 100000
DIM = 64
BATCH = 4096
SEQ = 200

_INFO = plsc.get_sparse_core_info()
_NC, _NS = _INFO.num_cores, _INFO.num_subcores
_NW = _NC * _NS
_BT = BATCH // _NW
_L = 16

_NB = 5
_GD = 3
_OUTER = SEQ // _NB


def _body(idx_hbm, table_hbm, out_hbm, idx_v, rows_v, tp_v, *sems):
    gsem = sems[:_NB]
    ssem = sems[_NB:]
    wid = lax.axis_index("s") * _NC + lax.axis_index("c")
    bt = wid
    pltpu.sync_copy(idx_hbm.at[pl.ds(0, SEQ), pl.ds(bt * _BT, _BT)], idx_v)

    def gather(s, rb):
        pltpu.async_copy(table_hbm.at[idx_v.at[s]], rows_v.at[rb], gsem[rb])

    def wait_gather(rb):
        pltpu.make_async_copy(
            table_hbm.at[idx_v.at[0]], rows_v.at[rb], gsem[rb]
        ).wait()

    def store(s, tb):
        for dt in range(DIM // 8):
            pltpu.async_copy(
                tp_v.at[tb, pl.ds(dt * 8, 8)],
                out_hbm.at[s, dt, bt],
                ssem[tb],
            )

    def wait_store(tb):
        for dt in range(DIM // 8):
            pltpu.make_async_copy(
                tp_v.at[tb, pl.ds(dt * 8, 8)],
                out_hbm.at[0, dt, bt],
                ssem[tb],
            ).wait()

    iota = lax.broadcasted_iota(jnp.int32, (_L,), 0)
    diag = [(iota + k) & (_L - 1) for k in range(_L)]
    iota64 = iota * DIM
    dk128 = [d * _BT for d in diag]
    zv = iota & 0

    def transpose(rb, tb):
        @plsc.parallel_loop(0, 32, unroll=4)
        def tile(t):
            d0 = (t // 8) * _L
            c0 = (t % 8) * _L
            bv = iota64 + (c0 * DIM + d0)
            wb = iota + (d0 * _BT + c0)
            for k in range(_L):
                v = plsc.load_gather(rows_v.at[rb], [zv, bv + diag[k]])
                plsc.store_scatter(tp_v.at[tb], [zv, wb + dk128[k]], v)

    for s in range(_GD):
        gather(s, s % _NB)

    def step(k, carry):
        for u in range(_NB):
            b = u
            bn = (u + _GD) % _NB
            s = k * _NB + u

            if u + _GD < _NB:
                gather(s + _GD, bn)
            else:
                @pl.when(k < _OUTER - 1)
                def _():
                    gather(s + _GD, bn)

            wait_gather(b)

            @pl.when(k > 0)
            def _():
                wait_store(b)

            transpose(b, b)
            store(s, b)
        return carry

    lax.fori_loop(0, _OUTER, step, 0)

    for tb in range(_NB):
        wait_store(tb)


def kernel(inputs, table):
    run = pl.kernel(
        _body,
        mesh=plsc.VectorSubcoreMesh(core_axis_name="c", subcore_axis_name="s"),
        out_type=jax.ShapeDtypeStruct(
            (SEQ, DIM // 8, _NW, 8, _BT), jnp.float32
        ),
        scratch_types=[
            pltpu.VMEM((SEQ, _BT), jnp.int32),
            pltpu.VMEM((_NB, _BT, DIM), jnp.float32),
            pltpu.VMEM((_NB, DIM, _BT), jnp.float32),
        ]
        + [pltpu.SemaphoreType.DMA] * (2 * _NB),
        compiler_params=pltpu.CompilerParams(
            use_tc_tiling_on_sc=False,
            needs_layout_passes=False,
            disable_bounds_checks=True,
        ),
    )
    out5 = run(jnp.transpose(inputs), table)
    return out5.transpose(2, 4, 0, 1, 3).reshape(BATCH, SEQ, DIM)


# speedup vs baseline: 13.5416x; 1.0054x over previous
import jax
import jax.numpy as jnp
from jax import lax
from jax.experimental import pallas as pl
from jax.experimental.pallas import tpu as pltpu
from jax.experimental.pallas import tpu_sc as plsc

VOCAB = 100000
DIM = 64
BATCH = 4096
SEQ = 200

_INFO = plsc.get_sparse_core_info()
_NC, _NS = _INFO.num_cores, _INFO.num_subcores
_NW = _NC * _NS
_BT = BATCH // _NW
_L = 16

_NB = 5
_GD = 3
_OUTER = SEQ // _NB


def _body(idx_hbm, table_hbm, out_hbm, idx_v, rows_v, tp_v, *sems):
    gsem = sems[:_NB]
    ssem = sems[_NB:]
    wid = lax.axis_index("s") * _NC + lax.axis_index("c")
    bt = wid
    pltpu.sync_copy(idx_hbm.at[pl.ds(0, SEQ), pl.ds(bt * _BT, _BT)], idx_v)

    def gather(s, rb):
        pltpu.async_copy(table_hbm.at[idx_v.at[s]], rows_v.at[rb], gsem[rb])

    def wait_gather(rb):
        pltpu.make_async_copy(
            table_hbm.at[idx_v.at[0]], rows_v.at[rb], gsem[rb]
        ).wait()

    def store(s, tb):
        pltpu.async_copy(
            tp_v.at[tb], out_hbm.at[s, pl.ds(0, DIM // 8), bt], ssem[tb]
        )

    def wait_store(tb):
        pltpu.make_async_copy(
            tp_v.at[tb], out_hbm.at[0, pl.ds(0, DIM // 8), bt], ssem[tb]
        ).wait()

    iota = lax.broadcasted_iota(jnp.int32, (_L,), 0)
    diag = [(iota + k) & (_L - 1) for k in range(_L)]
    iota64 = iota * DIM
    dk128 = [d * _BT for d in diag]
    zv = iota & 0

    def transpose(rb, tb):
        @plsc.parallel_loop(0, 32, unroll=4)
        def tile(t):
            d0 = (t // 8) * _L
            c0 = (t % 8) * _L
            bv = iota64 + (c0 * DIM + d0)
            wb = iota + (d0 * _BT + c0)
            for k in range(_L):
                v = plsc.load_gather(rows_v.at[rb], [zv, bv + diag[k]])
                plsc.store_scatter(tp_v.at[tb], [zv, zv, wb + dk128[k]], v)

    for s in range(_GD):
        gather(s, s % _NB)

    def step(k, carry):
        for u in range(_NB):
            b = u
            bn = (u + _GD) % _NB
            s = k * _NB + u

            if u + _GD < _NB:
                gather(s + _GD, bn)
            else:
                @pl.when(k < _OUTER - 1)
                def _():
                    gather(s + _GD, bn)

            wait_gather(b)

            @pl.when(k > 0)
            def _():
                wait_store(b)

            transpose(b, b)
            store(s, b)
        return carry

    lax.fori_loop(0, _OUTER, step, 0)

    for tb in range(_NB):
        wait_store(tb)


def kernel(inputs, table):
    run = pl.kernel(
        _body,
        mesh=plsc.VectorSubcoreMesh(core_axis_name="c", subcore_axis_name="s"),
        out_type=jax.ShapeDtypeStruct(
            (SEQ, DIM // 8, _NW, 8, _BT), jnp.float32
        ),
        scratch_types=[
            pltpu.VMEM((SEQ, _BT), jnp.int32),
            pltpu.VMEM((_NB, _BT, DIM), jnp.float32),
            pltpu.VMEM((_NB, DIM // 8, 8, _BT), jnp.float32),
        ]
        + [pltpu.SemaphoreType.DMA] * (2 * _NB),
        compiler_params=pltpu.CompilerParams(
            use_tc_tiling_on_sc=False,
            needs_layout_passes=False,
            disable_bounds_checks=True,
        ),
    )
    out5 = run(jnp.transpose(inputs), table)
    return out5.transpose(2, 4, 0, 1, 3).reshape(BATCH, SEQ, DIM)
